# baseline (speedup 1.0000x reference)
import jax
import jax.numpy as jnp
from jax.experimental import pallas as pl
from jax.experimental.pallas import tpu as pltpu

_B = 64
_TMEL = 2048
_NCH = 80
_TSRC = 512
_BB = 2
_NG = _B // _BB


def _loss_body(mt, mp, mq, mm3, mm2, pt, pp, et, ep, dt, ldp, sv,
               o_total, o_mel, o_post, o_dur, o_pitch, o_energy, acc, a1, a2):
    i = pl.program_id(0)

    @pl.when(i == 0)
    def _init():
        mmv = mm2[...]
        svv = sv[...]
        log_dur_trg = jnp.log(dt[...].astype(jnp.float32) + 1.0)
        acc[2] = jnp.sum(jnp.square(pp[...] - pt[...]) * mmv)
        acc[3] = jnp.sum(jnp.square(ep[...] - et[...]) * mmv)
        acc[4] = jnp.sum(jnp.square(ldp[...] - log_dur_trg) * svv)
        acc[5] = jnp.sum(mmv)
        acc[6] = jnp.sum(svv)
        a1[...] = jnp.zeros_like(a1)
        a2[...] = jnp.zeros_like(a2)

    t = mt[...]
    m = mm3[...]
    a1[...] += jnp.abs(mp[...] - t) * m
    a2[...] += jnp.abs(mq[...] - t) * m

    @pl.when(i == _NG - 1)
    def _fin():
        n_mel = acc[5]
        n_src = acc[6]
        mel_loss = jnp.sum(a1[...]) / (n_mel * _NCH)
        post_loss = jnp.sum(a2[...]) / (n_mel * _NCH)
        pitch_loss = acc[2] / n_mel
        energy_loss = acc[3] / n_mel
        dur_loss = acc[4] / n_src
        o_mel[0] = mel_loss
        o_post[0] = post_loss
        o_dur[0] = dur_loss
        o_pitch[0] = pitch_loss
        o_energy[0] = energy_loss
        o_total[0] = mel_loss + post_loss + dur_loss + pitch_loss + energy_loss


def kernel(mel_trg, dur_trg, pitch_trg, energy_trg, mel_pred,
           mel_postnet_pred, log_dur_pred, pitch_pred, energy_pred,
           src_mask, mel_mask):
    mt = jnp.transpose(mel_trg, (0, 2, 1))
    mp = jnp.transpose(mel_pred, (0, 2, 1))
    mq = jnp.transpose(mel_postnet_pred, (0, 2, 1))
    mm2 = mel_mask.astype(jnp.float32)
    mm3 = mm2.reshape(_B, 1, _TMEL)
    sv = jnp.logical_not(src_mask).astype(jnp.float32)

    mel_spec = pl.BlockSpec((_BB, _NCH, _TMEL), lambda i: (i, 0, 0))
    full = lambda shape: pl.BlockSpec(shape, lambda i: (0,) * len(shape))
    out_spec = pl.BlockSpec(memory_space=pltpu.SMEM)
    outs = pl.pallas_call(
        _loss_body,
        grid=(_NG,),
        in_specs=[
            mel_spec,
            mel_spec,
            mel_spec,
            pl.BlockSpec((_BB, 1, _TMEL), lambda i: (i, 0, 0)),
            full((_B, _TMEL)),
            full((_B, _TMEL)),
            full((_B, _TMEL)),
            full((_B, _TMEL)),
            full((_B, _TMEL)),
            full((_B, _TSRC)),
            full((_B, _TSRC)),
            full((_B, _TSRC)),
        ],
        out_specs=[out_spec] * 6,
        out_shape=[jax.ShapeDtypeStruct((1,), jnp.float32)] * 6,
        scratch_shapes=[pltpu.SMEM((8,), jnp.float32),
                        pltpu.VMEM((_BB, _NCH, _TMEL), jnp.float32),
                        pltpu.VMEM((_BB, _NCH, _TMEL), jnp.float32)],
    )(mt, mp, mq, mm3, mm2, pitch_trg, pitch_pred, energy_trg, energy_pred,
      dur_trg, log_dur_pred, sv)

    total, mel, post, dur, pitch, energy = [o[0] for o in outs]
    return (total, mel, post, dur, pitch, energy)


# speedup vs baseline: 1.1441x; 1.1441x over previous
import jax
import jax.numpy as jnp
from jax.experimental import pallas as pl
from jax.experimental.pallas import tpu as pltpu

_B = 64
_TMEL = 2048
_NCH = 80
_TSRC = 512
_BB = 4
_NG = _B // _BB


def _loss_body(mt, mp, mq, mm3, mm2, pt, pp, et, ep, dt, ldp, sv,
               o_total, o_mel, o_post, o_dur, o_pitch, o_energy, acc, a1, a2):
    i = pl.program_id(0)

    @pl.when(i == 0)
    def _init():
        for k in range(2, 7):
            acc[k] = 0.0
        a1[...] = jnp.zeros_like(a1)
        a2[...] = jnp.zeros_like(a2)

    t = mt[...]
    m = mm3[...]
    a1[...] += jnp.abs(mp[...] - t) * m
    a2[...] += jnp.abs(mq[...] - t) * m

    @pl.when(i % 2 == 0)
    def _small():
        mmv = mm2[...]
        svv = sv[...]
        log_dur_trg = jnp.log(dt[...].astype(jnp.float32) + 1.0)
        acc[2] = acc[2] + jnp.sum(jnp.square(pp[...] - pt[...]) * mmv)
        acc[3] = acc[3] + jnp.sum(jnp.square(ep[...] - et[...]) * mmv)
        acc[4] = acc[4] + jnp.sum(jnp.square(ldp[...] - log_dur_trg) * svv)
        acc[5] = acc[5] + jnp.sum(mmv)
        acc[6] = acc[6] + jnp.sum(svv)

    @pl.when(i == _NG - 1)
    def _fin():
        n_mel = acc[5]
        n_src = acc[6]
        mel_loss = jnp.sum(a1[...]) / (n_mel * _NCH)
        post_loss = jnp.sum(a2[...]) / (n_mel * _NCH)
        pitch_loss = acc[2] / n_mel
        energy_loss = acc[3] / n_mel
        dur_loss = acc[4] / n_src
        o_mel[0] = mel_loss
        o_post[0] = post_loss
        o_dur[0] = dur_loss
        o_pitch[0] = pitch_loss
        o_energy[0] = energy_loss
        o_total[0] = mel_loss + post_loss + dur_loss + pitch_loss + energy_loss


def kernel(mel_trg, dur_trg, pitch_trg, energy_trg, mel_pred,
           mel_postnet_pred, log_dur_pred, pitch_pred, energy_pred,
           src_mask, mel_mask):
    mt = jnp.transpose(mel_trg, (0, 2, 1))
    mp = jnp.transpose(mel_pred, (0, 2, 1))
    mq = jnp.transpose(mel_postnet_pred, (0, 2, 1))
    mm2 = mel_mask.astype(jnp.float32)
    mm3 = mm2.reshape(_B, 1, _TMEL)
    sv = jnp.logical_not(src_mask).astype(jnp.float32)

    mel_spec = pl.BlockSpec((_BB, _NCH, _TMEL), lambda i: (i, 0, 0))
    full = lambda shape: pl.BlockSpec(shape, lambda i: (0,) * len(shape))
    out_spec = pl.BlockSpec(memory_space=pltpu.SMEM)
    outs = pl.pallas_call(
        _loss_body,
        grid=(_NG,),
        in_specs=[
            mel_spec,
            mel_spec,
            mel_spec,
            pl.BlockSpec((_BB, 1, _TMEL), lambda i: (i, 0, 0)),
            pl.BlockSpec((8, _TMEL), lambda i: (i // 2, 0)),
            pl.BlockSpec((8, _TMEL), lambda i: (i // 2, 0)),
            pl.BlockSpec((8, _TMEL), lambda i: (i // 2, 0)),
            pl.BlockSpec((8, _TMEL), lambda i: (i // 2, 0)),
            pl.BlockSpec((8, _TMEL), lambda i: (i // 2, 0)),
            pl.BlockSpec((8, _TSRC), lambda i: (i // 2, 0)),
            pl.BlockSpec((8, _TSRC), lambda i: (i // 2, 0)),
            pl.BlockSpec((8, _TSRC), lambda i: (i // 2, 0)),
        ],
        out_specs=[out_spec] * 6,
        out_shape=[jax.ShapeDtypeStruct((1,), jnp.float32)] * 6,
        scratch_shapes=[pltpu.SMEM((8,), jnp.float32),
                        pltpu.VMEM((_BB, _NCH, _TMEL), jnp.float32),
                        pltpu.VMEM((_BB, _NCH, _TMEL), jnp.float32)],
    )(mt, mp, mq, mm3, mm2, pitch_trg, pitch_pred, energy_trg, energy_pred,
      dur_trg, log_dur_pred, sv)

    total, mel, post, dur, pitch, energy = [o[0] for o in outs]
    return (total, mel, post, dur, pitch, energy)
